# baseline (speedup 1.0000x reference)
import jax
import jax.numpy as jnp
from jax.experimental import pallas as pl
from jax.experimental.pallas import tpu as pltpu

N = 100000
D = 128
BUFROWS = 20000
NBUF = 2

SIZES = [5000, 20000, 20000, 20000, 20000, 10000, 2500, 2500]
OFFS = []
_off = 0
for _s in SIZES:
    OFFS.append(_off)
    _off += _s
assert _off == N
NCHUNKS = len(SIZES)


def _gcn_pipe(x_hbm, adj_ref, w_ref, b_ref, o_hbm, xbuf, obuf, m_ref, rsem, wsem):
    def rcopy(k, slot):
        sz = SIZES[k]
        return pltpu.make_async_copy(
            x_hbm.at[pl.ds(OFFS[k], sz), :],
            xbuf.at[slot, pl.ds(0, sz), :],
            rsem.at[slot],
        )

    def wcopy(k, slot):
        sz = SIZES[k]
        return pltpu.make_async_copy(
            obuf.at[slot, pl.ds(0, sz), :],
            o_hbm.at[pl.ds(OFFS[k], sz), :],
            wsem.at[slot],
        )

    for k in range(NBUF):
        rcopy(k, k).start()

    m_ref[...] = jax.lax.dot_general(
        w_ref[...], adj_ref[...],
        dimension_numbers=(((1,), (1,)), ((), ())),
        preferred_element_type=jnp.float32,
    )

    bias = b_ref[...]
    for k in range(NCHUNKS):
        slot = k % NBUF
        sz = SIZES[k]
        rcopy(k, slot).wait()
        if k >= NBUF:
            wcopy(k - NBUF, slot).wait()
        obuf[slot, :sz, :] = (
            jnp.dot(xbuf[slot, :sz, :], m_ref[...], preferred_element_type=jnp.float32)
            + bias
        )
        wcopy(k, slot).start()
        if k + NBUF < NCHUNKS:
            rcopy(k + NBUF, slot).start()

    for k in range(NCHUNKS - NBUF, NCHUNKS):
        wcopy(k, k % NBUF).wait()


def kernel(input, adj, weight, bias):
    bias2d = bias.reshape(1, D)
    return pl.pallas_call(
        _gcn_pipe,
        in_specs=[
            pl.BlockSpec(memory_space=pltpu.MemorySpace.HBM),
            pl.BlockSpec((D, D), lambda: (0, 0)),
            pl.BlockSpec((D, D), lambda: (0, 0)),
            pl.BlockSpec((1, D), lambda: (0, 0)),
        ],
        out_specs=pl.BlockSpec(memory_space=pltpu.MemorySpace.HBM),
        out_shape=jax.ShapeDtypeStruct((N, D), jnp.float32),
        scratch_shapes=[
            pltpu.VMEM((NBUF, BUFROWS, D), jnp.float32),
            pltpu.VMEM((NBUF, BUFROWS, D), jnp.float32),
            pltpu.VMEM((D, D), jnp.float32),
            pltpu.SemaphoreType.DMA((NBUF,)),
            pltpu.SemaphoreType.DMA((NBUF,)),
        ],
    )(input, adj, weight, bias2d)


# speedup vs baseline: 1.0420x; 1.0420x over previous
import jax
import jax.numpy as jnp
from jax.experimental import pallas as pl
from jax.experimental.pallas import tpu as pltpu

N = 100000
D = 128
BLK = 20000


def _gcn_kernel(x_ref, adj_ref, w_ref, b_ref, o_ref, m_ref):
    @pl.when(pl.program_id(0) == 0)
    def _():
        m_ref[...] = jax.lax.dot_general(
            w_ref[...], adj_ref[...],
            dimension_numbers=(((1,), (1,)), ((), ())),
            preferred_element_type=jnp.float32,
        )

    o_ref[...] = (
        jnp.dot(x_ref[...], m_ref[...], preferred_element_type=jnp.float32)
        + b_ref[...]
    )


def kernel(input, adj, weight, bias):
    bias2d = bias.reshape(1, D)
    return pl.pallas_call(
        _gcn_kernel,
        grid=(N // BLK,),
        in_specs=[
            pl.BlockSpec((BLK, D), lambda i: (i, 0)),
            pl.BlockSpec((D, D), lambda i: (0, 0)),
            pl.BlockSpec((D, D), lambda i: (0, 0)),
            pl.BlockSpec((1, D), lambda i: (0, 0)),
        ],
        out_specs=pl.BlockSpec((BLK, D), lambda i: (i, 0)),
        out_shape=jax.ShapeDtypeStruct((N, D), jnp.float32),
        scratch_shapes=[pltpu.VMEM((D, D), jnp.float32)],
    )(input, adj, weight, bias2d)
